# baseline (speedup 1.0000x reference)
import jax
import jax.numpy as jnp
from jax import lax
from jax.experimental import pallas as pl
from jax.experimental.pallas import tpu as pltpu

N_EXPERTS = 64
D_MODEL = 4096
SEQ = 8192
BATCH = 4
CHUNK = 512
N_CHUNKS = SEQ // CHUNK


def _router_body(x_ref, w_ref, ph_ref, ts_ref, ti_ref, coh_ref, acc_ref):
    c = pl.program_id(1)

    @pl.when(c == 0)
    def _init():
        acc_ref[...] = jnp.zeros_like(acc_ref)

    acc_ref[...] += jnp.sum(x_ref[0], axis=0, keepdims=True)

    @pl.when(c == N_CHUNKS - 1)
    def _finalize():
        pooled = acc_ref[...] * (1.0 / SEQ)
        amp = lax.dot_general(
            pooled, w_ref[...], (((1,), (1,)), ((), ())),
            preferred_element_type=jnp.float32,
        )
        ph = ph_ref[...]
        coh = jnp.abs(amp * (jnp.cos(ph) + jnp.sin(ph)))
        coh_ref[0] = coh

        iota = lax.broadcasted_iota(jnp.int32, (1, N_EXPERTS), 1)
        m1 = jnp.max(coh, axis=1, keepdims=True)
        i1 = jnp.min(jnp.where(coh == m1, iota, N_EXPERTS), axis=1, keepdims=True)
        coh2 = jnp.where(iota == i1, -1.0, coh)
        m2 = jnp.max(coh2, axis=1, keepdims=True)
        i2 = jnp.min(jnp.where(coh2 == m2, iota, N_EXPERTS), axis=1, keepdims=True)
        ts_ref[0] = jnp.where(iota == 0, m1, jnp.where(iota == 1, m2, 0.0))
        ti_ref[0] = jnp.where(iota == 0, i1, jnp.where(iota == 1, i2, 0))


def kernel(x, W, phase_angles, top_k):
    ph2 = phase_angles.reshape(1, N_EXPERTS)
    ts_pad, ti_pad, coherence = pl.pallas_call(
        _router_body,
        grid=(BATCH, N_CHUNKS),
        in_specs=[
            pl.BlockSpec((1, CHUNK, D_MODEL), lambda b, c: (b, c, 0)),
            pl.BlockSpec((N_EXPERTS, D_MODEL), lambda b, c: (0, 0)),
            pl.BlockSpec((1, N_EXPERTS), lambda b, c: (0, 0)),
        ],
        out_specs=[
            pl.BlockSpec((1, 1, N_EXPERTS), lambda b, c: (b, 0, 0)),
            pl.BlockSpec((1, 1, N_EXPERTS), lambda b, c: (b, 0, 0)),
            pl.BlockSpec((1, 1, N_EXPERTS), lambda b, c: (b, 0, 0)),
        ],
        out_shape=[
            jax.ShapeDtypeStruct((BATCH, 1, N_EXPERTS), jnp.float32),
            jax.ShapeDtypeStruct((BATCH, 1, N_EXPERTS), jnp.int32),
            jax.ShapeDtypeStruct((BATCH, 1, N_EXPERTS), jnp.float32),
        ],
        scratch_shapes=[pltpu.VMEM((1, D_MODEL), jnp.float32)],
        compiler_params=pltpu.CompilerParams(
            dimension_semantics=("parallel", "arbitrary"),
        ),
    )(x, W, ph2)
    delta = (jnp.asarray(top_k, jnp.int32) - 2).astype(jnp.float32)
    top_scores = ts_pad[:, 0, :2] + delta
    top_idx = ti_pad[:, 0, :2]
    return (top_scores, top_idx, coherence[:, 0, :])


# speedup vs baseline: 1.0037x; 1.0037x over previous
import jax
import jax.numpy as jnp
from jax import lax
from jax.experimental import pallas as pl
from jax.experimental.pallas import tpu as pltpu

N_EXPERTS = 64
D_MODEL = 4096
SEQ = 8192
BATCH = 4
CHUNK = 1024
N_CHUNKS = SEQ // CHUNK


def _router_body(x_ref, w_ref, ph_ref, ts_ref, ti_ref, coh_ref, acc_ref):
    c = pl.program_id(1)

    @pl.when(c == 0)
    def _init():
        acc_ref[...] = jnp.zeros_like(acc_ref)

    acc_ref[...] += jnp.sum(
        x_ref[0].reshape(CHUNK // 8, 8, D_MODEL), axis=0)

    @pl.when(c == N_CHUNKS - 1)
    def _finalize():
        pooled = jnp.sum(acc_ref[...], axis=0, keepdims=True) * (1.0 / SEQ)
        amp = lax.dot_general(
            pooled, w_ref[...], (((1,), (1,)), ((), ())),
            preferred_element_type=jnp.float32,
        )
        ph = ph_ref[...]
        coh = jnp.abs(amp * (jnp.cos(ph) + jnp.sin(ph)))
        coh_ref[0] = coh

        iota = lax.broadcasted_iota(jnp.int32, (1, N_EXPERTS), 1)
        m1 = jnp.max(coh, axis=1, keepdims=True)
        i1 = jnp.min(jnp.where(coh == m1, iota, N_EXPERTS), axis=1, keepdims=True)
        coh2 = jnp.where(iota == i1, -1.0, coh)
        m2 = jnp.max(coh2, axis=1, keepdims=True)
        i2 = jnp.min(jnp.where(coh2 == m2, iota, N_EXPERTS), axis=1, keepdims=True)
        ts_ref[0] = jnp.where(iota == 0, m1, jnp.where(iota == 1, m2, 0.0))
        ti_ref[0] = jnp.where(iota == 0, i1, jnp.where(iota == 1, i2, 0))


def kernel(x, W, phase_angles, top_k):
    ph2 = phase_angles.reshape(1, N_EXPERTS)
    ts_pad, ti_pad, coherence = pl.pallas_call(
        _router_body,
        grid=(BATCH, N_CHUNKS),
        in_specs=[
            pl.BlockSpec((1, CHUNK, D_MODEL), lambda b, c: (b, c, 0)),
            pl.BlockSpec((N_EXPERTS, D_MODEL), lambda b, c: (0, 0)),
            pl.BlockSpec((1, N_EXPERTS), lambda b, c: (0, 0)),
        ],
        out_specs=[
            pl.BlockSpec((1, 1, N_EXPERTS), lambda b, c: (b, 0, 0)),
            pl.BlockSpec((1, 1, N_EXPERTS), lambda b, c: (b, 0, 0)),
            pl.BlockSpec((1, 1, N_EXPERTS), lambda b, c: (b, 0, 0)),
        ],
        out_shape=[
            jax.ShapeDtypeStruct((BATCH, 1, N_EXPERTS), jnp.float32),
            jax.ShapeDtypeStruct((BATCH, 1, N_EXPERTS), jnp.int32),
            jax.ShapeDtypeStruct((BATCH, 1, N_EXPERTS), jnp.float32),
        ],
        scratch_shapes=[pltpu.VMEM((8, D_MODEL), jnp.float32)],
        compiler_params=pltpu.CompilerParams(
            dimension_semantics=("parallel", "arbitrary"),
        ),
    )(x, W, ph2)
    delta = (jnp.asarray(top_k, jnp.int32) - 2).astype(jnp.float32)
    top_scores = ts_pad[:, 0, :2] + delta
    top_idx = ti_pad[:, 0, :2]
    return (top_scores, top_idx, coherence[:, 0, :])
